# baseline (device time: 79648 ns/iter reference)
import jax
import jax.numpy as jnp
from jax import lax
from jax.experimental import pallas as pl
from jax.experimental.pallas import tpu as pltpu

B, S, H, Dh, Dr, D = 2, 256, 16, 64, 32, 1024
T = B * S
SCALE = (Dh + Dr) ** -0.5


def kernel(x, Wdkv, Wuk, Wuv, Wq, Wqr, Wkr, Wo):
    def body(x_ref, wdkv_ref, wuk_ref, wuv_ref, wq_ref, wqr_ref, wkr_ref,
             wo_ref, out_ref, kv_ref, recv_ref, o_ref, send_sem, recv_sem):
        my_x = lax.axis_index("x")
        my_y = lax.axis_index("y")
        my_z = lax.axis_index("z")
        partner = (1 - my_x, my_y, my_z)

        xf = x_ref[...].reshape(T, D)

        c = jnp.dot(xf, wdkv_ref[...], preferred_element_type=jnp.float32)
        kv_ref[0:T, :] = jnp.dot(c, wuk_ref[...],
                                 preferred_element_type=jnp.float32)
        kv_ref[T:2 * T, :] = jnp.dot(c, wuv_ref[...],
                                     preferred_element_type=jnp.float32)

        barrier_sem = pltpu.get_barrier_semaphore()
        pl.semaphore_signal(barrier_sem, inc=1, device_id=partner,
                            device_id_type=pl.DeviceIdType.MESH)
        pl.semaphore_wait(barrier_sem, 1)

        rdma = pltpu.make_async_remote_copy(
            src_ref=kv_ref,
            dst_ref=recv_ref,
            send_sem=send_sem,
            recv_sem=recv_sem,
            device_id=partner,
            device_id_type=pl.DeviceIdType.MESH,
        )
        rdma.start()

        q_all = jnp.dot(xf, wq_ref[...], preferred_element_type=jnp.float32)
        qr_all = jnp.dot(xf, wqr_ref[...], preferred_element_type=jnp.float32)
        kr_all = jnp.dot(xf, wkr_ref[...], preferred_element_type=jnp.float32)

        rdma.wait()
        k_all = kv_ref[0:T, :] + recv_ref[0:T, :]
        v_all = kv_ref[T:2 * T, :] + recv_ref[T:2 * T, :]

        for b in range(B):
            kr_b = kr_all[b * S:(b + 1) * S, :]
            for h in range(H):
                r0, r1 = b * S, (b + 1) * S
                c0, c1 = h * Dh, (h + 1) * Dh
                q = q_all[r0:r1, c0:c1]
                k = k_all[r0:r1, c0:c1]
                v = v_all[r0:r1, c0:c1]
                qr = qr_all[r0:r1, h * Dr:(h + 1) * Dr]
                s = (lax.dot_general(q, k, (((1,), (1,)), ((), ())),
                                     preferred_element_type=jnp.float32)
                     + lax.dot_general(qr, kr_b, (((1,), (1,)), ((), ())),
                                       preferred_element_type=jnp.float32)
                     ) * SCALE
                m = jnp.max(s, axis=-1, keepdims=True)
                e = jnp.exp(s - m)
                p = e / jnp.sum(e, axis=-1, keepdims=True)
                o_ref[r0:r1, c0:c1] = jnp.dot(
                    p, v, preferred_element_type=jnp.float32)

        out = jnp.dot(o_ref[...], wo_ref[...],
                      preferred_element_type=jnp.float32)
        out_ref[...] = out.reshape(B, S, D)

    return pl.pallas_call(
        body,
        out_shape=jax.ShapeDtypeStruct((B, S, D), jnp.float32),
        in_specs=[pl.BlockSpec(memory_space=pltpu.VMEM)] * 8,
        out_specs=pl.BlockSpec(memory_space=pltpu.VMEM),
        scratch_shapes=[
            pltpu.VMEM((2 * T, D), jnp.float32),
            pltpu.VMEM((2 * T, D), jnp.float32),
            pltpu.VMEM((T, D), jnp.float32),
            pltpu.SemaphoreType.DMA,
            pltpu.SemaphoreType.DMA,
        ],
        compiler_params=pltpu.CompilerParams(collective_id=0),
    )(x, Wdkv, Wuk, Wuv, Wq, Wqr, Wkr, Wo)


# device time: 44082 ns/iter; 1.8068x vs baseline; 1.8068x over previous
import jax
import jax.numpy as jnp
from jax import lax
from jax.experimental import pallas as pl
from jax.experimental.pallas import tpu as pltpu

B, S, H, Dh, Dr, D = 2, 256, 16, 64, 32, 1024
T = B * S
SCALE = (Dh + Dr) ** -0.5


def kernel(x, Wdkv, Wuk, Wuv, Wq, Wqr, Wkr, Wo):
    def body(x_ref, wdkv_ref, wuk_ref, wuv_ref, wq_ref, wqr_ref, wkr_ref,
             wo_ref, out_ref, c_ref, c_recv, wuk_recv, wuv_recv, o_ref,
             send_sems, recv_sems):
        my_x = lax.axis_index("x")
        my_y = lax.axis_index("y")
        my_z = lax.axis_index("z")
        partner = (1 - my_x, my_y, my_z)

        xf = x_ref[...].reshape(T, D)

        c_ref[...] = jnp.dot(xf, wdkv_ref[...],
                             preferred_element_type=jnp.float32)

        barrier_sem = pltpu.get_barrier_semaphore()
        pl.semaphore_signal(barrier_sem, inc=1, device_id=partner,
                            device_id_type=pl.DeviceIdType.MESH)
        pl.semaphore_wait(barrier_sem, 1)

        rdmas = []
        for i, (src, dst) in enumerate(
                [(c_ref, c_recv), (wuk_ref, wuk_recv), (wuv_ref, wuv_recv)]):
            r = pltpu.make_async_remote_copy(
                src_ref=src, dst_ref=dst,
                send_sem=send_sems.at[i], recv_sem=recv_sems.at[i],
                device_id=partner, device_id_type=pl.DeviceIdType.MESH,
            )
            r.start()
            rdmas.append(r)

        q_all = jnp.dot(xf, wq_ref[...], preferred_element_type=jnp.float32)
        qr_all = jnp.dot(xf, wqr_ref[...], preferred_element_type=jnp.float32)
        kr_all = jnp.dot(xf, wkr_ref[...], preferred_element_type=jnp.float32)

        for r in rdmas:
            r.wait()
        c_loc = c_ref[...]
        c_rem = c_recv[...]
        k_all = (jnp.dot(c_loc, wuk_ref[...],
                         preferred_element_type=jnp.float32)
                 + jnp.dot(c_rem, wuk_recv[...],
                           preferred_element_type=jnp.float32))
        v_all = (jnp.dot(c_loc, wuv_ref[...],
                         preferred_element_type=jnp.float32)
                 + jnp.dot(c_rem, wuv_recv[...],
                           preferred_element_type=jnp.float32))

        for b in range(B):
            kr_b = kr_all[b * S:(b + 1) * S, :]
            for h in range(H):
                r0, r1 = b * S, (b + 1) * S
                c0, c1 = h * Dh, (h + 1) * Dh
                q = q_all[r0:r1, c0:c1]
                k = k_all[r0:r1, c0:c1]
                v = v_all[r0:r1, c0:c1]
                qr = qr_all[r0:r1, h * Dr:(h + 1) * Dr]
                s = (lax.dot_general(q, k, (((1,), (1,)), ((), ())),
                                     preferred_element_type=jnp.float32)
                     + lax.dot_general(qr, kr_b, (((1,), (1,)), ((), ())),
                                       preferred_element_type=jnp.float32)
                     ) * SCALE
                m = jnp.max(s, axis=-1, keepdims=True)
                e = jnp.exp(s - m)
                p = e / jnp.sum(e, axis=-1, keepdims=True)
                o_ref[r0:r1, c0:c1] = jnp.dot(
                    p, v, preferred_element_type=jnp.float32)

        out = jnp.dot(o_ref[...], wo_ref[...],
                      preferred_element_type=jnp.float32)
        out_ref[...] = out.reshape(B, S, D)

    return pl.pallas_call(
        body,
        out_shape=jax.ShapeDtypeStruct((B, S, D), jnp.float32),
        in_specs=[pl.BlockSpec(memory_space=pltpu.VMEM)] * 8,
        out_specs=pl.BlockSpec(memory_space=pltpu.VMEM),
        scratch_shapes=[
            pltpu.VMEM((T, 64), jnp.float32),
            pltpu.VMEM((T, 64), jnp.float32),
            pltpu.VMEM((64, D), jnp.float32),
            pltpu.VMEM((64, D), jnp.float32),
            pltpu.VMEM((T, D), jnp.float32),
            pltpu.SemaphoreType.DMA((3,)),
            pltpu.SemaphoreType.DMA((3,)),
        ],
        compiler_params=pltpu.CompilerParams(collective_id=0),
    )(x, Wdkv, Wuk, Wuv, Wq, Wqr, Wkr, Wo)
